# baseline (speedup 1.0000x reference)
import jax
import jax.numpy as jnp
from jax import lax
from jax.experimental import pallas as pl
from jax.experimental.pallas import tpu as pltpu
from jax.experimental.pallas import tpu_sc as plsc

CARDINALITY = 1000000
EMBED_DIM = 64
BATCH = 16384

NUM_CORES = 2
NUM_SUBCORES = 16
NUM_WORKERS = NUM_CORES * NUM_SUBCORES
B_PER_W = BATCH // NUM_WORKERS
GROUP = 4
NGROUP = B_PER_W // GROUP
BLOCK = 64
LANES = 16
STAGE_W = EMBED_DIM * 128


def _gather_body(t3_hbm, idx_hbm, out_hbm, idx_v, stage_v, rout_v, sem0, sem1, osem):
    wid = lax.axis_index("s") * NUM_CORES + lax.axis_index("c")
    base = wid * B_PER_W

    pltpu.sync_copy(idx_hbm.at[pl.ds(base, B_PER_W)], idx_v)

    iota = lax.iota(jnp.int32, LANES)
    sems = (sem0, sem1)

    def fire(q, buf, tbv):
        s = sems[buf]
        for l in range(GROUP):
            tb = pl.multiple_of(tbv[q * GROUP + l], 128)
            for c_hi in range(8):
                pltpu.async_copy(
                    t3_hbm.at[c_hi, :, pl.ds(tb, 128)],
                    stage_v.at[buf, l * 8 + c_hi],
                    s,
                )

    def drain(buf):
        s = sems[buf]
        for _ in range(GROUP * 8):
            pltpu.make_async_copy(
                t3_hbm.at[0, :, pl.ds(0, 128)],
                stage_v.at[0, 0],
                s,
            ).wait()

    def do_pair(g2, carry):
        rv = idx_v[pl.ds(g2 * LANES, LANES)]
        kv = rv & 127
        tbv = (rv >> 7) << 7

        fire(0, 0, tbv)
        for q in range(LANES // GROUP):
            buf = q % 2
            if q + 1 < LANES // GROUP:
                fire(q + 1, 1 - buf, tbv)
            drain(buf)

            g = g2 * (LANES // GROUP) + q
            blk = g // (BLOCK // GROUP)
            rbuf = blk % 2
            for l in range(GROUP):
                lo = (g % (BLOCK // GROUP)) * GROUP + l
                kk = iota * 0 + kv[q * GROUP + l]
                for cg in range(EMBED_DIM // LANES):
                    cvec = cg * LANES + iota
                    vals = plsc.load_gather(
                        stage_v, [iota * 0 + buf, l * 8 + (cvec >> 3), cvec & 7, kk]
                    )
                    rout_v[rbuf, lo, pl.ds(cg * LANES, LANES)] = vals

            if q == LANES // GROUP - 1:
                @pl.when(g % (BLOCK // GROUP) == (BLOCK // GROUP) - 1)
                def _(blk=blk, rbuf=rbuf):
                    pltpu.async_copy(
                        rout_v.at[rbuf],
                        out_hbm.at[pl.ds(base + blk * BLOCK, BLOCK)],
                        osem,
                    ).wait()

        return carry

    lax.fori_loop(0, B_PER_W // LANES, do_pair, 0)


@jax.jit
def _sc_gather(table, idx):
    mesh = plsc.VectorSubcoreMesh(core_axis_name="c", subcore_axis_name="s")
    fn = pl.kernel(
        _gather_body,
        mesh=mesh,
        out_type=jax.ShapeDtypeStruct((BATCH, 2 * EMBED_DIM), jnp.float32),
        scratch_types=[
            pltpu.VMEM((B_PER_W,), jnp.int32),
            pltpu.VMEM((2, GROUP * 8, 8, 128), jnp.float32),
            pltpu.VMEM((2, BLOCK, 2 * EMBED_DIM), jnp.float32),
            pltpu.SemaphoreType.DMA,
            pltpu.SemaphoreType.DMA,
            pltpu.SemaphoreType.DMA,
        ],
        compiler_params=pltpu.CompilerParams(needs_layout_passes=False),
    )
    t3 = table.T.reshape(8, 8, CARDINALITY)
    out128 = fn(t3, idx)
    return out128[:, :EMBED_DIM]


def kernel(x, table):
    return _sc_gather(table, x.astype(jnp.int32))
